# Initial kernel scaffold; baseline (speedup 1.0000x reference)
import jax
import jax.numpy as jnp
from jax.experimental import pallas as pl


def kernel(x, edge_index, batch, W1, b1, W2, b2, W3, b3, Wc, bc):
    raise NotImplementedError("write your pallas kernel here")


# speedup vs baseline: 22.3878x; 22.3878x over previous
import functools

import jax
import jax.numpy as jnp
from jax import lax
from jax.experimental import pallas as pl
from jax.experimental.pallas import tpu as pltpu
from jax.experimental.pallas import tpu_sc as plsc

N = 100000
E = 1600000
G = 64
D_IN = 6
DH = 32
DOUT = 6

NC = 2
NS = 16
NHALF = N // NC
K = 800
NCHUNK = E // K
T = NCHUNK // NS

TRASH = 1024
STRIPE1 = 3136
DEG_PAD = NS * STRIPE1
ACC1_ROWS = DEG_PAD + TRASH
ACC2_ROWS = NHALF + TRASH
STRIPE2 = 3128
QOUT = 400
NQ = NHALF // QOUT

_mesh = plsc.VectorSubcoreMesh(core_axis_name="c", subcore_axis_name="s")


def _localize(d16, base):
    local = d16 - base
    ok = (local >= 0) & (local < NHALF)
    trash = NHALF + (d16 & (TRASH - 1))
    return jnp.where(ok, local, trash)


@functools.partial(
    pl.kernel,
    mesh=_mesh,
    out_type=[
        jax.ShapeDtypeStruct((NC * DEG_PAD,), jnp.float32),
        jax.ShapeDtypeStruct((NC * E,), jnp.int32),
    ],
    scratch_types=[
        pltpu.VMEM((K,), jnp.int32),
        pltpu.VMEM((K,), jnp.int32),
        pltpu.VMEM((K,), jnp.float32),
        pltpu.VMEM_SHARED((ACC1_ROWS,), jnp.float32),
    ],
)
def _sc_deg(dst_hbm, deg_hbm, dloc_hbm, dbuf, lbuf, ones_v, acc):
    c = lax.axis_index("c")
    s = lax.axis_index("s")
    base = c * NHALF

    def zfill(i, _):
        ones_v[pl.ds(i * 16, 16)] = jnp.zeros((16,), jnp.float32)
        return 0
    lax.fori_loop(0, K // 16, zfill, 0)
    z0 = s * STRIPE1
    for k0 in (0, K, 2 * K, 3 * K):
        n = min(K, STRIPE1 - k0)
        pltpu.sync_copy(ones_v.at[pl.ds(0, n)], acc.at[pl.ds(z0 + k0, n)])
    def ofill(i, _):
        ones_v[pl.ds(i * 16, 16)] = jnp.full((16,), 1.0, jnp.float32)
        return 0
    lax.fori_loop(0, K // 16, ofill, 0)
    plsc.subcore_barrier()

    def chunk(t, _):
        off = (t * NS + s) * K
        pltpu.sync_copy(dst_hbm.at[pl.ds(off, K)], dbuf)

        def clamp(j, _):
            d16 = dbuf[pl.ds(j * 16, 16)]
            lbuf[pl.ds(j * 16, 16)] = _localize(d16, base)
            return 0
        lax.fori_loop(0, K // 16, clamp, 0)

        pltpu.sync_copy(ones_v, acc.at[lbuf], add=True)
        pltpu.sync_copy(lbuf, dloc_hbm.at[pl.ds(c * E + off, K)])
        return 0
    lax.fori_loop(0, T, chunk, 0)
    plsc.subcore_barrier()

    for k0 in (0, K, 2 * K, 3 * K):
        n = min(K, STRIPE1 - k0)
        pltpu.sync_copy(acc.at[pl.ds(z0 + k0, n)], ones_v.at[pl.ds(0, n)])
        pltpu.sync_copy(ones_v.at[pl.ds(0, n)],
                        deg_hbm.at[pl.ds(c * DEG_PAD + z0 + k0, n)])


@functools.partial(
    pl.kernel,
    mesh=_mesh,
    compiler_params=pltpu.CompilerParams(use_tc_tiling_on_sc=False),
    out_type=jax.ShapeDtypeStruct((N, DH), jnp.float32),
    scratch_types=[
        pltpu.VMEM((K,), jnp.int32),
        pltpu.VMEM((K,), jnp.int32),
        pltpu.VMEM((K, DH), jnp.float32),
        pltpu.VMEM_SHARED((ACC2_ROWS, DH), jnp.float32),
        pltpu.SemaphoreType.DMA,
    ],
)
def _sc_scatter(hp_hbm, src_hbm, dloc_hbm, acc_hbm, sbuf, dbuf, rows, acc, sem):
    c = lax.axis_index("c")
    s = lax.axis_index("s")

    def zfill(i, _):
        r = i // 2
        q = (i % 2) * 16
        rows[r, pl.ds(q, 16)] = jnp.zeros((16,), jnp.float32)
        return 0
    lax.fori_loop(0, 2 * K, zfill, 0)
    z0 = s * STRIPE2
    for k0 in (0, K, 2 * K, 3 * K):
        n = min(K, STRIPE2 - k0)
        pltpu.sync_copy(rows.at[pl.ds(0, n)], acc.at[pl.ds(z0 + k0, n)])
    plsc.subcore_barrier()

    def chunk(t, _):
        off = (t * NS + s) * K
        pltpu.sync_copy(src_hbm.at[pl.ds(off, K)], sbuf)
        pltpu.sync_copy(dloc_hbm.at[pl.ds(c * E + off, K)], dbuf)
        pltpu.async_copy(hp_hbm.at[sbuf], rows, sem).wait()
        pltpu.sync_copy(rows, acc.at[dbuf], add=True)
        return 0
    lax.fori_loop(0, T, chunk, 0)
    plsc.subcore_barrier()

    def cp(j, _):
        r0 = (s + j * NS) * QOUT
        pltpu.sync_copy(acc.at[pl.ds(r0, QOUT)], rows.at[pl.ds(0, QOUT)])
        pltpu.sync_copy(rows.at[pl.ds(0, QOUT)],
                        acc_hbm.at[pl.ds(c * NHALF + r0, QOUT)])
        return 0
    lax.fori_loop(0, (NQ - s + NS - 1) // NS, cp, 0)


B = 2000
GRID = N // B


def _tc1_body(x_ref, cnt_ref, w_ref, o_ref):
    dinv = lax.rsqrt(cnt_ref[...] + 1.0)
    h = jnp.dot(x_ref[...], w_ref[...], preferred_element_type=jnp.float32)
    o_ref[...] = h * dinv


def _tc1(x, cnt, W1):
    return pl.pallas_call(
        _tc1_body,
        grid=(GRID,),
        in_specs=[
            pl.BlockSpec((B, D_IN), lambda i: (i, 0)),
            pl.BlockSpec((B, 1), lambda i: (i, 0)),
            pl.BlockSpec((D_IN, DH), lambda i: (0, 0)),
        ],
        out_specs=pl.BlockSpec((B, DH), lambda i: (i, 0)),
        out_shape=jax.ShapeDtypeStruct((N, DH), jnp.float32),
    )(x, cnt, W1)


def _tcs_body(acc_ref, hp_ref, cnt_ref, b_ref, w_ref, o_ref):
    dinv = lax.rsqrt(cnt_ref[...] + 1.0)
    f = jnp.tanh(dinv * (acc_ref[...] + hp_ref[...]) + b_ref[...])
    o_ref[...] = jnp.dot(f, w_ref[...],
                         preferred_element_type=jnp.float32) * dinv


def _tc_stage(acc, hp, cnt, b, W):
    return pl.pallas_call(
        _tcs_body,
        grid=(GRID,),
        in_specs=[
            pl.BlockSpec((B, DH), lambda i: (i, 0)),
            pl.BlockSpec((B, DH), lambda i: (i, 0)),
            pl.BlockSpec((B, 1), lambda i: (i, 0)),
            pl.BlockSpec((1, DH), lambda i: (0, 0)),
            pl.BlockSpec((DH, DH), lambda i: (0, 0)),
        ],
        out_specs=pl.BlockSpec((B, DH), lambda i: (i, 0)),
        out_shape=jax.ShapeDtypeStruct((N, DH), jnp.float32),
    )(acc, hp, cnt, b, W)


def _tc4_body(acc_ref, hp_ref, cnt_ref, b_ref, batch_ref, wc_ref, bc_ref,
              o_ref, sums, cnts):
    i = pl.program_id(0)

    @pl.when(i == 0)
    def _():
        sums[...] = jnp.zeros_like(sums)
        cnts[...] = jnp.zeros_like(cnts)

    dinv = lax.rsqrt(cnt_ref[...] + 1.0)
    f = jnp.tanh(dinv * (acc_ref[...] + hp_ref[...]) + b_ref[...])
    onehot = (lax.broadcasted_iota(jnp.int32, (B, G), 1)
              == batch_ref[...]).astype(jnp.float32)
    sums[...] += lax.dot_general(onehot, f, (((0,), (0,)), ((), ())),
                                 preferred_element_type=jnp.float32)
    cnts[...] += jnp.sum(onehot, axis=0).reshape(G, 1)

    @pl.when(i == pl.num_programs(0) - 1)
    def _():
        pooled = sums[...] / jnp.maximum(cnts[...], 1.0)
        o_ref[...] = jnp.dot(pooled, wc_ref[...],
                             preferred_element_type=jnp.float32) + bc_ref[...]


def _tc4(acc, hp, cnt, b, batch, Wc, bc):
    return pl.pallas_call(
        _tc4_body,
        grid=(GRID,),
        in_specs=[
            pl.BlockSpec((B, DH), lambda i: (i, 0)),
            pl.BlockSpec((B, DH), lambda i: (i, 0)),
            pl.BlockSpec((B, 1), lambda i: (i, 0)),
            pl.BlockSpec((1, DH), lambda i: (0, 0)),
            pl.BlockSpec((B, 1), lambda i: (i, 0)),
            pl.BlockSpec((DH, DOUT), lambda i: (0, 0)),
            pl.BlockSpec((1, DOUT), lambda i: (0, 0)),
        ],
        out_specs=pl.BlockSpec((G, DOUT), lambda i: (0, 0)),
        out_shape=jax.ShapeDtypeStruct((G, DOUT), jnp.float32),
        scratch_shapes=[
            pltpu.VMEM((G, DH), jnp.float32),
            pltpu.VMEM((G, 1), jnp.float32),
        ],
    )(acc, hp, cnt, b, batch, Wc, bc)


def kernel(x, edge_index, batch, W1, b1, W2, b2, W3, b3, Wc, bc):
    src = edge_index[0]
    dst = edge_index[1]

    deg_pad, dst_loc = _sc_deg(dst)
    cnt = jnp.concatenate(
        [deg_pad[:NHALF], deg_pad[DEG_PAD:DEG_PAD + NHALF]]).reshape(N, 1)

    h1p = _tc1(x, cnt, W1)
    acc1 = _sc_scatter(h1p, src, dst_loc)
    h2p = _tc_stage(acc1, h1p, cnt, b1.reshape(1, DH), W2)
    acc2 = _sc_scatter(h2p, src, dst_loc)
    h3p = _tc_stage(acc2, h2p, cnt, b2.reshape(1, DH), W3)
    acc3 = _sc_scatter(h3p, src, dst_loc)
    return _tc4(acc3, h3p, cnt, b3.reshape(1, DH), batch.reshape(N, 1), Wc,
                bc.reshape(1, DOUT))


# speedup vs baseline: 23.5748x; 1.0530x over previous
import functools

import jax
import jax.numpy as jnp
from jax import lax
from jax.experimental import pallas as pl
from jax.experimental.pallas import tpu as pltpu
from jax.experimental.pallas import tpu_sc as plsc

N = 100000
E = 1600000
G = 64
D_IN = 6
DH = 32
DOUT = 6

NC = 2
NS = 16
NHALF = N // NC
K = 800
NCHUNK = E // K
T = NCHUNK // NS

TRASH = 1024
STRIPE1 = 3136
DEG_PAD = NS * STRIPE1
ACC1_ROWS = DEG_PAD + TRASH
ACC2_ROWS = NHALF + TRASH
STRIPE2 = 3128
QOUT = 400
NQ = NHALF // QOUT

_mesh = plsc.VectorSubcoreMesh(core_axis_name="c", subcore_axis_name="s")


def _localize(d16, base):
    local = d16 - base
    ok = (local >= 0) & (local < NHALF)
    trash = NHALF + (d16 & (TRASH - 1))
    return jnp.where(ok, local, trash)


@functools.partial(
    pl.kernel,
    mesh=_mesh,
    out_type=[
        jax.ShapeDtypeStruct((NC * DEG_PAD,), jnp.float32),
        jax.ShapeDtypeStruct((NC * E,), jnp.int32),
    ],
    scratch_types=[
        pltpu.VMEM((K,), jnp.int32),
        pltpu.VMEM((K,), jnp.int32),
        pltpu.VMEM((K,), jnp.float32),
        pltpu.VMEM_SHARED((ACC1_ROWS,), jnp.float32),
    ],
)
def _sc_deg(dst_hbm, deg_hbm, dloc_hbm, dbuf, lbuf, ones_v, acc):
    c = lax.axis_index("c")
    s = lax.axis_index("s")
    base = c * NHALF

    def zfill(i, _):
        ones_v[pl.ds(i * 16, 16)] = jnp.zeros((16,), jnp.float32)
        return 0
    lax.fori_loop(0, K // 16, zfill, 0)
    z0 = s * STRIPE1
    for k0 in (0, K, 2 * K, 3 * K):
        n = min(K, STRIPE1 - k0)
        pltpu.sync_copy(ones_v.at[pl.ds(0, n)], acc.at[pl.ds(z0 + k0, n)])
    def ofill(i, _):
        ones_v[pl.ds(i * 16, 16)] = jnp.full((16,), 1.0, jnp.float32)
        return 0
    lax.fori_loop(0, K // 16, ofill, 0)
    plsc.subcore_barrier()

    def chunk(t, _):
        off = (t * NS + s) * K
        pltpu.sync_copy(dst_hbm.at[pl.ds(off, K)], dbuf)

        def clamp(j, _):
            d16 = dbuf[pl.ds(j * 16, 16)]
            lbuf[pl.ds(j * 16, 16)] = _localize(d16, base)
            return 0
        lax.fori_loop(0, K // 16, clamp, 0)

        pltpu.sync_copy(ones_v, acc.at[lbuf], add=True)
        pltpu.sync_copy(lbuf, dloc_hbm.at[pl.ds(c * E + off, K)])
        return 0
    lax.fori_loop(0, T, chunk, 0)
    plsc.subcore_barrier()

    for k0 in (0, K, 2 * K, 3 * K):
        n = min(K, STRIPE1 - k0)
        pltpu.sync_copy(acc.at[pl.ds(z0 + k0, n)], ones_v.at[pl.ds(0, n)])
        pltpu.sync_copy(ones_v.at[pl.ds(0, n)],
                        deg_hbm.at[pl.ds(c * DEG_PAD + z0 + k0, n)])


K2 = 400
NCHUNK2 = E // K2
T2 = NCHUNK2 // NS
GROUPS = T2 // 2


@functools.partial(
    pl.kernel,
    mesh=_mesh,
    compiler_params=pltpu.CompilerParams(use_tc_tiling_on_sc=False),
    out_type=jax.ShapeDtypeStruct((N, DH), jnp.float32),
    scratch_types=[
        pltpu.VMEM((K2,), jnp.int32),
        pltpu.VMEM((K2,), jnp.int32),
        pltpu.VMEM((K2,), jnp.int32),
        pltpu.VMEM((K2,), jnp.int32),
        pltpu.VMEM((K2, DH), jnp.float32),
        pltpu.VMEM((K2, DH), jnp.float32),
        pltpu.SemaphoreType.DMA,
        pltpu.SemaphoreType.DMA,
        pltpu.SemaphoreType.DMA,
        pltpu.SemaphoreType.DMA,
        pltpu.SemaphoreType.DMA,
        pltpu.SemaphoreType.DMA,
        pltpu.VMEM_SHARED((ACC2_ROWS, DH), jnp.float32),
    ],
)
def _sc_scatter(hp_hbm, src_hbm, dloc_hbm, acc_hbm,
                sbuf0, sbuf1, dbuf0, dbuf1, rows0, rows1,
                isem0, isem1, gsem0, gsem1, ssem0, ssem1, acc):
    c = lax.axis_index("c")
    s = lax.axis_index("s")

    def zfill(i, _):
        r = i // 2
        q = (i % 2) * 16
        rows0[r, pl.ds(q, 16)] = jnp.zeros((16,), jnp.float32)
        return 0
    lax.fori_loop(0, 2 * K2, zfill, 0)
    z0 = s * STRIPE2
    for k0 in range(0, STRIPE2, K2):
        n = min(K2, STRIPE2 - k0)
        pltpu.sync_copy(rows0.at[pl.ds(0, n)], acc.at[pl.ds(z0 + k0, n)])
    plsc.subcore_barrier()

    slots = ((sbuf0, dbuf0, rows0, isem0, gsem0, ssem0),
             (sbuf1, dbuf1, rows1, isem1, gsem1, ssem1))

    def issue_idx(t, slot):
        sb, db, _, isem, _, _ = slot
        off = (t * NS + s) * K2
        pltpu.async_copy(src_hbm.at[pl.ds(off, K2)], sb, isem)
        pltpu.async_copy(dloc_hbm.at[pl.ds(c * E + off, K2)], db, isem)

    def run_slot(i, t, slot):
        sb, db, rw, isem, gsem, ssem = slot
        @pl.when(i > 0)
        def _():
            pltpu.make_async_copy(rw, acc.at[db], ssem).wait()
        issue_idx(t, slot)
        pltpu.make_async_copy(src_hbm.at[pl.ds(0, K2)], sb, isem).wait()
        pltpu.make_async_copy(src_hbm.at[pl.ds(0, K2)], db, isem).wait()
        pltpu.async_copy(hp_hbm.at[sb], rw, gsem).wait()
        pltpu.async_copy(rw, acc.at[db], ssem, add=True)

    def group(i, _):
        run_slot(i, 2 * i, slots[0])
        run_slot(i, 2 * i + 1, slots[1])
        return 0
    lax.fori_loop(0, GROUPS, group, 0)
    pltpu.make_async_copy(rows0, acc.at[dbuf0], ssem0).wait()
    pltpu.make_async_copy(rows1, acc.at[dbuf1], ssem1).wait()
    plsc.subcore_barrier()

    def cp(j, _):
        r0 = (s + j * NS) * QOUT
        pltpu.sync_copy(acc.at[pl.ds(r0, QOUT)], rows0.at[pl.ds(0, QOUT)])
        pltpu.sync_copy(rows0.at[pl.ds(0, QOUT)],
                        acc_hbm.at[pl.ds(c * NHALF + r0, QOUT)])
        return 0
    lax.fori_loop(0, (NQ - s + NS - 1) // NS, cp, 0)


B = 2000
GRID = N // B


def _tc1_body(x_ref, cnt_ref, w_ref, o_ref):
    dinv = lax.rsqrt(cnt_ref[...] + 1.0)
    h = jnp.dot(x_ref[...], w_ref[...], preferred_element_type=jnp.float32)
    o_ref[...] = h * dinv


def _tc1(x, cnt, W1):
    return pl.pallas_call(
        _tc1_body,
        grid=(GRID,),
        in_specs=[
            pl.BlockSpec((B, D_IN), lambda i: (i, 0)),
            pl.BlockSpec((B, 1), lambda i: (i, 0)),
            pl.BlockSpec((D_IN, DH), lambda i: (0, 0)),
        ],
        out_specs=pl.BlockSpec((B, DH), lambda i: (i, 0)),
        out_shape=jax.ShapeDtypeStruct((N, DH), jnp.float32),
    )(x, cnt, W1)


def _tcs_body(acc_ref, hp_ref, cnt_ref, b_ref, w_ref, o_ref):
    dinv = lax.rsqrt(cnt_ref[...] + 1.0)
    f = jnp.tanh(dinv * (acc_ref[...] + hp_ref[...]) + b_ref[...])
    o_ref[...] = jnp.dot(f, w_ref[...],
                         preferred_element_type=jnp.float32) * dinv


def _tc_stage(acc, hp, cnt, b, W):
    return pl.pallas_call(
        _tcs_body,
        grid=(GRID,),
        in_specs=[
            pl.BlockSpec((B, DH), lambda i: (i, 0)),
            pl.BlockSpec((B, DH), lambda i: (i, 0)),
            pl.BlockSpec((B, 1), lambda i: (i, 0)),
            pl.BlockSpec((1, DH), lambda i: (0, 0)),
            pl.BlockSpec((DH, DH), lambda i: (0, 0)),
        ],
        out_specs=pl.BlockSpec((B, DH), lambda i: (i, 0)),
        out_shape=jax.ShapeDtypeStruct((N, DH), jnp.float32),
    )(acc, hp, cnt, b, W)


def _tc4_body(acc_ref, hp_ref, cnt_ref, b_ref, batch_ref, wc_ref, bc_ref,
              o_ref, sums, cnts):
    i = pl.program_id(0)

    @pl.when(i == 0)
    def _():
        sums[...] = jnp.zeros_like(sums)
        cnts[...] = jnp.zeros_like(cnts)

    dinv = lax.rsqrt(cnt_ref[...] + 1.0)
    f = jnp.tanh(dinv * (acc_ref[...] + hp_ref[...]) + b_ref[...])
    onehot = (lax.broadcasted_iota(jnp.int32, (B, G), 1)
              == batch_ref[...]).astype(jnp.float32)
    sums[...] += lax.dot_general(onehot, f, (((0,), (0,)), ((), ())),
                                 preferred_element_type=jnp.float32)
    cnts[...] += jnp.sum(onehot, axis=0).reshape(G, 1)

    @pl.when(i == pl.num_programs(0) - 1)
    def _():
        pooled = sums[...] / jnp.maximum(cnts[...], 1.0)
        o_ref[...] = jnp.dot(pooled, wc_ref[...],
                             preferred_element_type=jnp.float32) + bc_ref[...]


def _tc4(acc, hp, cnt, b, batch, Wc, bc):
    return pl.pallas_call(
        _tc4_body,
        grid=(GRID,),
        in_specs=[
            pl.BlockSpec((B, DH), lambda i: (i, 0)),
            pl.BlockSpec((B, DH), lambda i: (i, 0)),
            pl.BlockSpec((B, 1), lambda i: (i, 0)),
            pl.BlockSpec((1, DH), lambda i: (0, 0)),
            pl.BlockSpec((B, 1), lambda i: (i, 0)),
            pl.BlockSpec((DH, DOUT), lambda i: (0, 0)),
            pl.BlockSpec((1, DOUT), lambda i: (0, 0)),
        ],
        out_specs=pl.BlockSpec((G, DOUT), lambda i: (0, 0)),
        out_shape=jax.ShapeDtypeStruct((G, DOUT), jnp.float32),
        scratch_shapes=[
            pltpu.VMEM((G, DH), jnp.float32),
            pltpu.VMEM((G, 1), jnp.float32),
        ],
    )(acc, hp, cnt, b, batch, Wc, bc)


def kernel(x, edge_index, batch, W1, b1, W2, b2, W3, b3, Wc, bc):
    src = edge_index[0]
    dst = edge_index[1]

    deg_pad, dst_loc = _sc_deg(dst)
    cnt = jnp.concatenate(
        [deg_pad[:NHALF], deg_pad[DEG_PAD:DEG_PAD + NHALF]]).reshape(N, 1)

    h1p = _tc1(x, cnt, W1)
    acc1 = _sc_scatter(h1p, src, dst_loc)
    h2p = _tc_stage(acc1, h1p, cnt, b1.reshape(1, DH), W2)
    acc2 = _sc_scatter(h2p, src, dst_loc)
    h3p = _tc_stage(acc2, h2p, cnt, b2.reshape(1, DH), W3)
    acc3 = _sc_scatter(h3p, src, dst_loc)
    return _tc4(acc3, h3p, cnt, b3.reshape(1, DH), batch.reshape(N, 1), Wc,
                bc.reshape(1, DOUT))
